# baseline (speedup 1.0000x reference)
import jax
import numpy as np
import jax.numpy as jnp
from jax import lax
from jax.experimental import pallas as pl
from jax.experimental.pallas import tpu as pltpu

_B = 32
_N = 8192
_NQ = 1024
_SUB, _LANE = 64, 128
_FLIP = np.int32(-2147483648)


def _rotl(x, r):
    return lax.shift_left(x, np.int32(r)) | lax.shift_right_logical(
        x, np.int32(32 - r))


def _threefry_bits(k0, k1, idx):
    rot = ((13, 15, 26, 6), (17, 29, 16, 24))
    ks = (k0, k1, k0 ^ k1 ^ np.int32(0x1BD11BDA))
    x0 = jnp.full_like(idx, 0) + ks[0]
    x1 = idx + ks[1]
    for i in range(5):
        for r in rot[i % 2]:
            x0 = x0 + x1
            x1 = _rotl(x1, r) ^ x0
        x0 = x0 + ks[(i + 1) % 3]
        x1 = x1 + ks[(i + 2) % 3] + np.int32(i + 1)
    return x0 ^ x1


def _bit(r, c, v):
    if v < _LANE:
        return (c & v) != 0
    return (r & (v // _LANE)) != 0


def _partner(x, j, upper):
    ax, dj = (1, j) if j < _LANE else (0, j // _LANE)
    return jnp.where(upper, jnp.roll(x, dj, axis=ax), jnp.roll(x, -dj, axis=ax))


def _passes():
    k = 2
    while k <= _N:
        j = k // 2
        while j >= 1:
            yield k, j
            j //= 2
        k *= 2


def _sort_kernel(sk_ref, out_ref):
    r = lax.broadcasted_iota(jnp.int32, (_SUB, _LANE), 0)
    c = lax.broadcasted_iota(jnp.int32, (_SUB, _LANE), 1)
    idx = r * _LANE + c

    b = _threefry_bits(sk_ref[0, 0, 0], sk_ref[0, 0, 1], idx) ^ _FLIP
    v = idx
    for k, j in _passes():
        upper = _bit(r, c, j)
        take_min = _bit(r, c, k) == upper
        pb = _partner(b, j, upper)
        pv = _partner(v, j, upper)
        swap = (take_min & (pb < b)) | (~take_min & (pb > b))
        b = jnp.where(swap, pb, b)
        v = jnp.where(swap, pv, v)

    b = _threefry_bits(sk_ref[0, 0, 2], sk_ref[0, 0, 3], idx) ^ _FLIP
    t = idx
    for k, j in _passes():
        upper = _bit(r, c, j)
        take_min = _bit(r, c, k) == upper
        pb = _partner(b, j, upper)
        pv = _partner(v, j, upper)
        pt = _partner(t, j, upper)
        p_less = (pb < b) | ((pb == b) & (pt < t))
        swap = take_min == p_less
        b = jnp.where(swap, pb, b)
        v = jnp.where(swap, pv, v)
        t = jnp.where(swap, pt, t)

    out_ref[0] = v[: _NQ // _LANE, :]


def _subkey_table():
    keys = jax.random.split(jax.random.key(42), _B)
    s1 = jax.vmap(jax.random.split)(keys)
    s2 = jax.vmap(jax.random.split)(s1[:, 0])
    d1 = jax.random.key_data(s1[:, 1])
    d2 = jax.random.key_data(s2[:, 1])
    return lax.bitcast_convert_type(
        jnp.concatenate([d1, d2], axis=1), jnp.int32).reshape(_B, 1, 4)


def kernel(points):
    del points
    sk = _subkey_table()
    out = pl.pallas_call(
        _sort_kernel,
        grid=(_B,),
        in_specs=[pl.BlockSpec((1, 1, 4), lambda i: (i, 0, 0),
                               memory_space=pltpu.SMEM)],
        out_specs=pl.BlockSpec((1, _NQ // _LANE, _LANE), lambda i: (i, 0, 0)),
        out_shape=jax.ShapeDtypeStruct((_B, _NQ // _LANE, _LANE), jnp.int32),
    )(sk)
    return out.reshape(_B, _NQ)


# speedup vs baseline: 2.4486x; 2.4486x over previous
import jax
import numpy as np
import jax.numpy as jnp
from jax import lax
from jax.experimental import pallas as pl
from jax.experimental.pallas import tpu as pltpu

_B = 32
_N = 8192
_NQ = 1024
_SUB, _LANE = 64, 128
_RPS = 4
_FLIP = np.int32(-2147483648)


def _rotl(x, r):
    return lax.shift_left(x, np.int32(r)) | lax.shift_right_logical(
        x, np.int32(32 - r))


def _threefry_bits(k0, k1, idx):
    rot = ((13, 15, 26, 6), (17, 29, 16, 24))
    ks = (k0, k1, k0 ^ k1 ^ np.int32(0x1BD11BDA))
    x0 = jnp.full_like(idx, 0) + ks[0]
    x1 = idx + ks[1]
    for i in range(5):
        for r in rot[i % 2]:
            x0 = x0 + x1
            x1 = _rotl(x1, r) ^ x0
        x0 = x0 + ks[(i + 1) % 3]
        x1 = x1 + ks[(i + 2) % 3] + np.int32(i + 1)
    return x0 ^ x1


def _bit(r, c, v):
    if v < _LANE:
        return (c & v) != 0
    return (r & (v // _LANE)) != 0


def _partner(x, j, upper):
    ax, dj = (1, j) if j < _LANE else (0, j // _LANE)
    return jnp.where(upper, jnp.roll(x, dj, axis=ax), jnp.roll(x, -dj, axis=ax))


def _passes():
    k = 2
    while k <= _N:
        j = k // 2
        while j >= 1:
            yield k, j
            j //= 2
        k *= 2


def _sort_kernel(sk_ref, out_ref):
    r = lax.broadcasted_iota(jnp.int32, (_SUB, _LANE), 0)
    c = lax.broadcasted_iota(jnp.int32, (_SUB, _LANE), 1)
    idx = r * _LANE + c

    bs = [_threefry_bits(sk_ref[q, 0, 0], sk_ref[q, 0, 1], idx) ^ _FLIP
          for q in range(_RPS)]
    vs = [idx] * _RPS
    for k, j in _passes():
        upper = _bit(r, c, j)
        take_min = _bit(r, c, k) == upper
        for q in range(_RPS):
            b, v = bs[q], vs[q]
            pb = _partner(b, j, upper)
            nb = jnp.where(take_min, jnp.minimum(b, pb), jnp.maximum(b, pb))
            swap = nb != b
            bs[q] = nb
            vs[q] = jnp.where(swap, _partner(v, j, upper), v)

    ws = [lax.shift_left(idx, np.int32(13)) | vs[q] for q in range(_RPS)]
    bs = [_threefry_bits(sk_ref[q, 0, 2], sk_ref[q, 0, 3], idx) ^ _FLIP
          for q in range(_RPS)]
    for k, j in _passes():
        upper = _bit(r, c, j)
        take_min = _bit(r, c, k) == upper
        for q in range(_RPS):
            b, w = bs[q], ws[q]
            pb = _partner(b, j, upper)
            pw = _partner(w, j, upper)
            p_less = (pb < b) | ((pb == b) & (pw < w))
            swap = take_min == p_less
            bs[q] = jnp.where(swap, pb, b)
            ws[q] = jnp.where(swap, pw, w)

    for q in range(_RPS):
        out_ref[q] = ws[q][: _NQ // _LANE, :] & np.int32(_N - 1)


def _subkey_table():
    keys = jax.random.split(jax.random.key(42), _B)
    s1 = jax.vmap(jax.random.split)(keys)
    s2 = jax.vmap(jax.random.split)(s1[:, 0])
    d1 = jax.random.key_data(s1[:, 1])
    d2 = jax.random.key_data(s2[:, 1])
    return lax.bitcast_convert_type(
        jnp.concatenate([d1, d2], axis=1), jnp.int32).reshape(_B, 1, 4)


def kernel(points):
    del points
    sk = _subkey_table()
    out = pl.pallas_call(
        _sort_kernel,
        grid=(_B // _RPS,),
        in_specs=[pl.BlockSpec((_RPS, 1, 4), lambda i: (i, 0, 0),
                               memory_space=pltpu.SMEM)],
        out_specs=pl.BlockSpec((_RPS, _NQ // _LANE, _LANE),
                               lambda i: (i, 0, 0)),
        out_shape=jax.ShapeDtypeStruct((_B, _NQ // _LANE, _LANE), jnp.int32),
    )(sk)
    return out.reshape(_B, _NQ)
